# baseline (speedup 1.0000x reference)
import jax
import jax.numpy as jnp
from jax.experimental import pallas as pl
from jax.experimental.pallas import tpu as pltpu

_D = 2048
_E = 64
_H = 8
_F = _H * _E
_FA = 6 * _E
_SEG = (0, 0, 1, 2, 2, 3, 4, 5)
_TOPK = (8, 8, 8, 8, 8, 8, 4, 4)
_TILE = 256


def _prep_kernel(emb_ref, bd_ref, seg_ref):
    emb = emb_ref[...]
    norm = jnp.sqrt(jnp.sum(emb * emb, axis=1, keepdims=True))
    embn = emb / (norm + 1e-8)
    bd_ref[...] = jnp.zeros((_F, _F), dtype=jnp.float32)
    seg_ref[...] = jnp.zeros((_F, _H), dtype=jnp.float32)
    for h in range(_H):
        s = _SEG[h] * _E
        c = h * _E
        bd_ref[c:c + _E, c:c + _E] = embn[s:s + _E, :].T
        seg_ref[c:c + _E, h:h + 1] = jnp.ones((_E, 1), dtype=jnp.float32)


def _route_kernel(x_ref, wa_ref, ba_ref, wf_ref, bf_ref, wr_ref, br_ref,
                  bd_ref, seg_ref, out_ref):
    xt = x_ref[0]
    pa = jnp.dot(xt, wa_ref[...],
                 preferred_element_type=jnp.float32) + ba_ref[...]
    pf = jnp.dot(xt, wf_ref[...],
                 preferred_element_type=jnp.float32) + bf_ref[...]
    pr = jnp.dot(xt, wr_ref[...],
                 preferred_element_type=jnp.float32) + br_ref[...]
    logits = (
        jnp.dot(pa, bd_ref[0:_FA, :], preferred_element_type=jnp.float32)
        + jnp.dot(pf, bd_ref[_FA:_FA + _E, :],
                  preferred_element_type=jnp.float32)
        + jnp.dot(pr, bd_ref[_FA + _E:_F, :],
                  preferred_element_type=jnp.float32))
    e = jnp.exp(logits)
    zs = jnp.dot(e, seg_ref[...],
                 preferred_element_type=jnp.float32)
    for h in range(_H):
        c = h * _E
        eh = e[:, c:c + _E]
        active = eh
        acc = jnp.zeros((_TILE, 1), dtype=jnp.float32)
        for _ in range(_TOPK[h]):
            mx = jnp.max(active, axis=1, keepdims=True)
            acc = acc + mx
            active = jnp.where(active == mx, -1.0, active)
        sparse = eh - jnp.maximum(active, 0.0)
        inv = 1.0 / (acc + 1e-8 * zs[:, h:h + 1])
        out_ref[h, 0, :, :] = sparse * inv


def kernel(x, W_all, b_all, W_fk, b_fk, W_rk, b_rk, neuron_emb):
    B, S, D = x.shape
    bd, seg = pl.pallas_call(
        _prep_kernel,
        out_shape=(
            jax.ShapeDtypeStruct((_F, _F), jnp.float32),
            jax.ShapeDtypeStruct((_F, _H), jnp.float32),
        ),
    )(neuron_emb)

    out = pl.pallas_call(
        _route_kernel,
        grid=(B, S // _TILE),
        in_specs=[
            pl.BlockSpec((1, _TILE, _D), lambda b, i: (b, i, 0)),
            pl.BlockSpec((_D, _FA), lambda b, i: (0, 0)),
            pl.BlockSpec((1, _FA), lambda b, i: (0, 0)),
            pl.BlockSpec((_D, _E), lambda b, i: (0, 0)),
            pl.BlockSpec((1, _E), lambda b, i: (0, 0)),
            pl.BlockSpec((_D, _E), lambda b, i: (0, 0)),
            pl.BlockSpec((1, _E), lambda b, i: (0, 0)),
            pl.BlockSpec((_F, _F), lambda b, i: (0, 0)),
            pl.BlockSpec((_F, _H), lambda b, i: (0, 0)),
        ],
        out_specs=pl.BlockSpec((_H, 1, _TILE, _E), lambda b, i: (0, b, i, 0)),
        out_shape=jax.ShapeDtypeStruct((_H, B, S, _E), jnp.float32),
    )(x, W_all, b_all.reshape(1, _FA), W_fk, b_fk.reshape(1, _E),
      W_rk, b_rk.reshape(1, _E), bd, seg)
    return out


# speedup vs baseline: 1.1358x; 1.1358x over previous
import jax
import jax.numpy as jnp
from jax.experimental import pallas as pl
from jax.experimental.pallas import tpu as pltpu

_D = 2048
_E = 64
_H = 8
_F = _H * _E
_SEG = (0, 0, 1, 2, 2, 3, 4, 5)
_TOPK = (8, 8, 8, 8, 8, 8, 4, 4)
_TILE = 512


def _prep_kernel(emb_ref, bd_ref, seg_ref):
    emb = emb_ref[...]
    norm = jnp.sqrt(jnp.sum(emb * emb, axis=1, keepdims=True))
    embn = emb / (norm + 1e-8)
    bd_ref[...] = jnp.zeros((_F, _F), dtype=jnp.float32)
    seg_ref[...] = jnp.zeros((_F, _H), dtype=jnp.float32)
    for h in range(_H):
        s = _SEG[h] * _E
        c = h * _E
        bd_ref[c:c + _E, c:c + _E] = embn[s:s + _E, :].T
        seg_ref[c:c + _E, h:h + 1] = jnp.ones((_E, 1), dtype=jnp.float32)


def _route_kernel(x_ref, w_ref, b_ref, bd_ref, seg_ref, out_ref):
    proj = jnp.dot(x_ref[...], w_ref[...],
                   preferred_element_type=jnp.float32) + b_ref[...]
    logits = jnp.dot(proj, bd_ref[...],
                     preferred_element_type=jnp.float32)
    e = jnp.exp(logits)
    zs = jnp.dot(e, seg_ref[...],
                 preferred_element_type=jnp.float32)
    for h in range(_H):
        c = h * _E
        eh = e[:, c:c + _E]
        active = eh
        acc = jnp.zeros((_TILE, 1), dtype=jnp.float32)
        for _ in range(_TOPK[h]):
            mx = jnp.max(active, axis=1, keepdims=True)
            acc = acc + mx
            active = jnp.where(active == mx, -1.0, active)
        sparse = eh - jnp.maximum(active, 0.0)
        inv = 1.0 / (acc + 1e-8 * zs[:, h:h + 1])
        out_ref[h, :, :] = sparse * inv


def kernel(x, W_all, b_all, W_fk, b_fk, W_rk, b_rk, neuron_emb):
    B, S, D = x.shape
    tokens = B * S
    x2 = x.reshape(tokens, D)
    w_cat = jnp.concatenate([W_all, W_fk, W_rk], axis=1)
    b_cat = jnp.concatenate([b_all, b_fk, b_rk]).reshape(1, _F)

    bd, seg = pl.pallas_call(
        _prep_kernel,
        out_shape=(
            jax.ShapeDtypeStruct((_F, _F), jnp.float32),
            jax.ShapeDtypeStruct((_F, _H), jnp.float32),
        ),
    )(neuron_emb)

    n_tiles = tokens // _TILE
    out = pl.pallas_call(
        _route_kernel,
        grid=(n_tiles,),
        in_specs=[
            pl.BlockSpec((_TILE, _D), lambda i: (i, 0)),
            pl.BlockSpec((_D, _F), lambda i: (0, 0)),
            pl.BlockSpec((1, _F), lambda i: (0, 0)),
            pl.BlockSpec((_F, _F), lambda i: (0, 0)),
            pl.BlockSpec((_F, _H), lambda i: (0, 0)),
        ],
        out_specs=pl.BlockSpec((_H, _TILE, _E), lambda i: (0, i, 0)),
        out_shape=jax.ShapeDtypeStruct((_H, tokens, _E), jnp.float32),
    )(x2, w_cat, b_cat, bd, seg)
    return out.reshape(_H, B, S, _E)


# speedup vs baseline: 1.2372x; 1.0893x over previous
import jax
import jax.numpy as jnp
from jax.experimental import pallas as pl
from jax.experimental.pallas import tpu as pltpu

_D = 2048
_E = 64
_H = 8
_F = _H * _E
_SEG = (0, 0, 1, 2, 2, 3, 4, 5)
_TOPK = (8, 8, 8, 8, 8, 8, 4, 4)
_TILE = 1024


def _prep_kernel(emb_ref, bd_ref, seg_ref):
    emb = emb_ref[...]
    norm = jnp.sqrt(jnp.sum(emb * emb, axis=1, keepdims=True))
    embn = emb / (norm + 1e-8)
    bd_ref[...] = jnp.zeros((_F, _F), dtype=jnp.float32)
    seg_ref[...] = jnp.zeros((_F, _H), dtype=jnp.float32)
    for h in range(_H):
        s = _SEG[h] * _E
        c = h * _E
        bd_ref[c:c + _E, c:c + _E] = embn[s:s + _E, :].T
        seg_ref[c:c + _E, h:h + 1] = jnp.ones((_E, 1), dtype=jnp.float32)


def _route_kernel(x_ref, w_ref, b_ref, bd_ref, seg_ref, out_ref):
    proj = jnp.dot(x_ref[...], w_ref[...],
                   preferred_element_type=jnp.float32) + b_ref[...]
    logits = jnp.dot(proj, bd_ref[...],
                     preferred_element_type=jnp.float32)
    e = jnp.exp(logits)
    zs = jnp.dot(e, seg_ref[...],
                 preferred_element_type=jnp.float32)
    for h in range(_H):
        c = h * _E
        eh = e[:, c:c + _E]
        active = eh
        acc = jnp.zeros((_TILE, 1), dtype=jnp.float32)
        for _ in range(_TOPK[h]):
            mx = jnp.max(active, axis=1, keepdims=True)
            acc = acc + mx
            active = jnp.where(active == mx, -1.0, active)
        sparse = eh - jnp.maximum(active, 0.0)
        inv = 1.0 / (acc + 1e-8 * zs[:, h:h + 1])
        out_ref[h, :, :] = sparse * inv


def kernel(x, W_all, b_all, W_fk, b_fk, W_rk, b_rk, neuron_emb):
    B, S, D = x.shape
    tokens = B * S
    x2 = x.reshape(tokens, D)
    w_cat = jnp.concatenate([W_all, W_fk, W_rk], axis=1)
    b_cat = jnp.concatenate([b_all, b_fk, b_rk]).reshape(1, _F)

    bd, seg = pl.pallas_call(
        _prep_kernel,
        out_shape=(
            jax.ShapeDtypeStruct((_F, _F), jnp.float32),
            jax.ShapeDtypeStruct((_F, _H), jnp.float32),
        ),
    )(neuron_emb)

    n_tiles = tokens // _TILE
    out = pl.pallas_call(
        _route_kernel,
        grid=(n_tiles,),
        in_specs=[
            pl.BlockSpec((_TILE, _D), lambda i: (i, 0)),
            pl.BlockSpec((_D, _F), lambda i: (0, 0)),
            pl.BlockSpec((1, _F), lambda i: (0, 0)),
            pl.BlockSpec((_F, _F), lambda i: (0, 0)),
            pl.BlockSpec((_F, _H), lambda i: (0, 0)),
        ],
        out_specs=pl.BlockSpec((_H, _TILE, _E), lambda i: (0, i, 0)),
        out_shape=jax.ShapeDtypeStruct((_H, tokens, _E), jnp.float32),
    )(x2, w_cat, b_cat, bd, seg)
    return out.reshape(_H, B, S, _E)
